# baseline (speedup 1.0000x reference)
import functools

import jax
import jax.numpy as jnp
from jax import lax
from jax.experimental import pallas as pl
from jax.experimental.pallas import tpu as pltpu
from jax.experimental.pallas import tpu_sc as plsc

N, E, D, R, G = 10000, 320000, 128, 4, 64
HID, F2, OUT = 32, 64, 10

NC, NS = 2, 16
NW = NC * NS
CH = 128
EPT = E // NW
NBUF = 4
NCHUNK = 80
EPT_PAD = NCHUNK * CH
E_PAD = EPT_PAD * NW
NPAD = 10112
RPT = NPAD // NS
DEGW = 16
NB = 1000
NBLK = N // NB


def _edge_pass_body(with_deg, F, table_hbm, gidx_hbm, didx_hbm, *refs):
    if with_deg:
        (acc_out, deg_out, gidx_v, didx_v, r0, r1, r2, r3, acc_sh,
         gs0, gs1, gs2, gs3, ss0, ss1, ss2, ss3, ones_v, deg_sh, dsem) = refs
    else:
        (acc_out, gidx_v, didx_v, r0, r1, r2, r3, acc_sh,
         gs0, gs1, gs2, gs3, ss0, ss1, ss2, ss3) = refs
    rows = (r0, r1, r2, r3)
    gsem = (gs0, gs1, gs2, gs3)
    ssem = (ss0, ss1, ss2, ss3)
    c = lax.axis_index("c")
    s = lax.axis_index("s")
    wid = s * NC + c

    pltpu.sync_copy(gidx_hbm.at[wid], gidx_v)
    pltpu.sync_copy(didx_hbm.at[wid], didx_v)

    zv = jnp.zeros((16,), jnp.float32)

    def zrow(i, _):
        for b in range(NBUF):
            for k in range(F // 16):
                rows[b][i, pl.ds(k * 16, 16)] = zv
        return 0

    lax.fori_loop(0, CH, zrow, 0)
    base = s * RPT
    for off in range(0, RPT, CH):
        ln = min(CH, RPT - off)
        pltpu.sync_copy(rows[0].at[pl.ds(0, ln)], acc_sh.at[pl.ds(base + off, ln)])

    if with_deg:
        def zrow2(i, _):
            ones_v[i, pl.ds(0, 16)] = zv
            return 0

        lax.fori_loop(0, CH, zrow2, 0)
        for off in range(0, RPT, CH):
            ln = min(CH, RPT - off)
            pltpu.sync_copy(ones_v.at[pl.ds(0, ln)], deg_sh.at[pl.ds(base + off, ln)])
        ov = jnp.ones((16,), jnp.float32)

        def orow(i, _):
            ones_v[i, pl.ds(0, 16)] = ov
            return 0

        lax.fori_loop(0, CH, orow, 0)

    plsc.subcore_barrier()

    def g_wait(b):
        pltpu.make_async_copy(table_hbm.at[gidx_v.at[0]], rows[b],
                              gsem[b]).wait()

    def do_chunk(j, b):
        g_wait(b)
        pltpu.sync_copy(rows[b], acc_sh.at[didx_v.at[j]], add=True)
        if with_deg:
            pltpu.sync_copy(ones_v, deg_sh.at[didx_v.at[j]], add=True)

    for b in range(NBUF - 1):
        pltpu.async_copy(table_hbm.at[gidx_v.at[b]], rows[b], gsem[b])

    def quad(j0, _):
        for b in range(NBUF):
            j = j0 * NBUF + b
            b2 = (b + NBUF - 1) % NBUF
            pltpu.async_copy(table_hbm.at[gidx_v.at[j + NBUF - 1]],
                             rows[b2], gsem[b2])
            do_chunk(j, b)
        return 0

    lax.fori_loop(0, (NCHUNK - NBUF) // NBUF, quad, 0)

    for t in range(NBUF):
        j = NCHUNK - NBUF + t
        b = j % NBUF
        if t == 0:
            b2 = (b + NBUF - 1) % NBUF
            pltpu.async_copy(table_hbm.at[gidx_v.at[NCHUNK - 1]],
                             rows[b2], gsem[b2])
        do_chunk(j, b)

    plsc.subcore_barrier()

    pltpu.sync_copy(acc_sh.at[pl.ds(base, RPT)], acc_out.at[c, pl.ds(base, RPT)])
    if with_deg:
        pltpu.sync_copy(deg_sh.at[pl.ds(base, RPT)], deg_out.at[c, pl.ds(base, RPT)])


def _make_edge_pass(F, with_deg):
    mesh = plsc.VectorSubcoreMesh(core_axis_name="c", subcore_axis_name="s")
    out_type = [jax.ShapeDtypeStruct((NC, NPAD, F), jnp.float32)]
    scratch = [
        pltpu.VMEM((NCHUNK, CH), jnp.int32),
        pltpu.VMEM((NCHUNK, CH), jnp.int32),
    ] + [pltpu.VMEM((CH, F), jnp.float32) for _ in range(NBUF)] + [
        pltpu.VMEM_SHARED((NPAD, F), jnp.float32),
    ] + [pltpu.SemaphoreType.DMA for _ in range(2 * NBUF)]
    if with_deg:
        out_type.append(jax.ShapeDtypeStruct((NC, NPAD, DEGW), jnp.float32))
        scratch += [
            pltpu.VMEM((CH, DEGW), jnp.float32),
            pltpu.VMEM_SHARED((NPAD, DEGW), jnp.float32),
            pltpu.SemaphoreType.DMA,
        ]
    return pl.kernel(
        functools.partial(_edge_pass_body, with_deg, F),
        out_type=out_type,
        mesh=mesh,
        scratch_types=scratch,
        compiler_params=pltpu.CompilerParams(use_tc_tiling_on_sc=False),
    )


_edge_pass_rgcn = _make_edge_pass(HID, True)
_edge_pass_gcn = _make_edge_pass(F2, False)


def _tc_a_body(x_ref, wrel_ref, wself_ref, brg_ref, hrel_ref, selfp_ref):
    xb = x_ref[...]
    for r in range(R):
        hrel_ref[r] = jnp.dot(xb, wrel_ref[r], preferred_element_type=jnp.float32)
    selfp_ref[...] = (
        jnp.dot(xb, wself_ref[...], preferred_element_type=jnp.float32)
        + brg_ref[...]
    )


_tc_a = pl.pallas_call(
    _tc_a_body,
    grid=(NBLK,),
    in_specs=[
        pl.BlockSpec((NB, D), lambda i: (i, 0)),
        pl.BlockSpec((R, D, HID), lambda i: (0, 0, 0)),
        pl.BlockSpec((D, HID), lambda i: (0, 0)),
        pl.BlockSpec((1, HID), lambda i: (0, 0)),
    ],
    out_specs=[
        pl.BlockSpec((R, NB, HID), lambda i: (0, i, 0)),
        pl.BlockSpec((NB, HID), lambda i: (i, 0)),
    ],
    out_shape=[
        jax.ShapeDtypeStruct((R, N, HID), jnp.float32),
        jax.ShapeDtypeStruct((N, HID), jnp.float32),
    ],
)


def _tc_c_body(acc_ref, selfp_ref, degp_ref, w1_ref, g1_ref, dinv_ref):
    h0 = jnp.maximum(acc_ref[0] + acc_ref[1] + selfp_ref[...], 0.0)
    deg = degp_ref[0] + degp_ref[1] + 1.0
    dinv = lax.rsqrt(deg)
    dinv_ref[...] = dinv
    g1_ref[...] = dinv[:, 0:1] * jnp.dot(
        h0, w1_ref[...], preferred_element_type=jnp.float32)


_tc_c = pl.pallas_call(
    _tc_c_body,
    grid=(NBLK,),
    in_specs=[
        pl.BlockSpec((NC, NB, HID), lambda i: (0, i, 0)),
        pl.BlockSpec((NB, HID), lambda i: (i, 0)),
        pl.BlockSpec((NC, NB, DEGW), lambda i: (0, i, 0)),
        pl.BlockSpec((HID, F2), lambda i: (0, 0)),
    ],
    out_specs=[
        pl.BlockSpec((NB, F2), lambda i: (i, 0)),
        pl.BlockSpec((NB, DEGW), lambda i: (i, 0)),
    ],
    out_shape=[
        jax.ShapeDtypeStruct((N, F2), jnp.float32),
        jax.ShapeDtypeStruct((N, DEGW), jnp.float32),
    ],
)


def _tc_e_body(acc_ref, g1_ref, dinv_ref, w3_ref, b1_ref, g2_ref):
    dinv1 = dinv_ref[:, 0:1]
    out1 = jnp.maximum(
        dinv1 * (acc_ref[0] + acc_ref[1] + g1_ref[...]) + b1_ref[...], 0.0)
    g2_ref[...] = dinv1 * jnp.dot(
        out1, w3_ref[...], preferred_element_type=jnp.float32)


_tc_e = pl.pallas_call(
    _tc_e_body,
    grid=(NBLK,),
    in_specs=[
        pl.BlockSpec((NC, NB, F2), lambda i: (0, i, 0)),
        pl.BlockSpec((NB, F2), lambda i: (i, 0)),
        pl.BlockSpec((NB, DEGW), lambda i: (i, 0)),
        pl.BlockSpec((F2, F2), lambda i: (0, 0)),
        pl.BlockSpec((1, F2), lambda i: (0, 0)),
    ],
    out_specs=pl.BlockSpec((NB, F2), lambda i: (i, 0)),
    out_shape=jax.ShapeDtypeStruct((N, F2), jnp.float32),
)


def _tc_g_body(acc_ref, g2_ref, dinv_ref, batch_ref, b3_ref, wl1_ref, bl1_ref,
               wl2_ref, bl2_ref, out_ref, pool_ref, cnt_ref):
    i = pl.program_id(0)

    @pl.when(i == 0)
    def _init():
        pool_ref[...] = jnp.zeros_like(pool_ref)
        cnt_ref[...] = jnp.zeros_like(cnt_ref)

    dinv1 = dinv_ref[:, 0:1]
    h2 = dinv1 * (acc_ref[0] + acc_ref[1] + g2_ref[...]) + b3_ref[...]
    onehot = (batch_ref[...] == lax.broadcasted_iota(
        jnp.int32, (NB, G), 1)).astype(jnp.float32)
    dn = (((0,), (0,)), ((), ()))
    pool_ref[...] += lax.dot_general(
        onehot, h2, dn, preferred_element_type=jnp.float32)
    cnt_ref[...] += lax.dot_general(
        onehot, jnp.ones((NB, 1), jnp.float32), dn,
        preferred_element_type=jnp.float32)

    @pl.when(i == NBLK - 1)
    def _fin():
        pooled = pool_ref[...] / jnp.maximum(cnt_ref[...], 1.0)
        o1 = jnp.dot(pooled, wl1_ref[...],
                     preferred_element_type=jnp.float32) + bl1_ref[...]
        o2 = jnp.dot(o1, wl2_ref[...],
                     preferred_element_type=jnp.float32) + bl2_ref[...]
        m = jnp.max(o2, axis=-1, keepdims=True)
        ex = jnp.exp(o2 - m)
        out_ref[...] = ex / jnp.sum(ex, axis=-1, keepdims=True)


_tc_g = pl.pallas_call(
    _tc_g_body,
    grid=(NBLK,),
    in_specs=[
        pl.BlockSpec((NC, NB, F2), lambda i: (0, i, 0)),
        pl.BlockSpec((NB, F2), lambda i: (i, 0)),
        pl.BlockSpec((NB, DEGW), lambda i: (i, 0)),
        pl.BlockSpec((NB, 1), lambda i: (i, 0)),
        pl.BlockSpec((1, F2), lambda i: (0, 0)),
        pl.BlockSpec((F2, 32), lambda i: (0, 0)),
        pl.BlockSpec((1, 32), lambda i: (0, 0)),
        pl.BlockSpec((32, OUT), lambda i: (0, 0)),
        pl.BlockSpec((1, OUT), lambda i: (0, 0)),
    ],
    out_specs=pl.BlockSpec((G, OUT), lambda i: (0, 0)),
    out_shape=jax.ShapeDtypeStruct((G, OUT), jnp.float32),
    scratch_shapes=[
        pltpu.VMEM((G, F2), jnp.float32),
        pltpu.VMEM((G, 1), jnp.float32),
    ],
)


def kernel(x, edge_index, edge_attr, batch, W_rel, W_self, b_rgcn,
           W1, b1, W3, b3, Wl1, bl1, Wl2, bl2):
    src = edge_index[0]
    dst = edge_index[1]
    pad = E_PAD - E
    zpad = jnp.zeros((pad,), jnp.int32)
    gidx1 = jnp.concatenate([edge_attr * N + src, zpad]).reshape(NW, NCHUNK, CH)
    src_p = jnp.concatenate([src, zpad]).reshape(NW, NCHUNK, CH)
    didx = jnp.concatenate([dst, jnp.full((pad,), N, jnp.int32)]
                           ).reshape(NW, NCHUNK, CH)

    hrel, selfp = _tc_a(x, W_rel, W_self, b_rgcn.reshape(1, HID))
    acc0, degp = _edge_pass_rgcn(hrel.reshape(R * N, HID), gidx1, didx)
    g1, dinv = _tc_c(acc0, selfp, degp, W1)
    acc1, = _edge_pass_gcn(g1, src_p, didx)
    g2 = _tc_e(acc1, g1, dinv, W3, b1.reshape(1, F2))
    acc2, = _edge_pass_gcn(g2, src_p, didx)
    return _tc_g(acc2, g2, dinv, batch.reshape(N, 1), b3.reshape(1, F2),
                 Wl1, bl1.reshape(1, 32), Wl2, bl2.reshape(1, OUT))


# speedup vs baseline: 1.4947x; 1.4947x over previous
import functools

import jax
import jax.numpy as jnp
from jax import lax
from jax.experimental import pallas as pl
from jax.experimental.pallas import tpu as pltpu
from jax.experimental.pallas import tpu_sc as plsc

N, E, D, R, G = 10000, 320000, 128, 4, 64
HID, F2, OUT = 32, 64, 10

NC, NS = 2, 16
NW = NC * NS
CH = 128
EPT = E // NW
NBUF = 8
NCHUNK = 80
EPT_PAD = NCHUNK * CH
E_PAD = EPT_PAD * NW
NPAD = 10112
RPT = NPAD // NS
DEGW = 16
NB = 1000
NBLK = N // NB


def _edge_pass_body(with_deg, F, trows, staged, table_hbm, gidx_hbm,
                    didx_hbm, *refs):
    if with_deg:
        (acc_out, deg_out, gidx_v, didx_v, rows_v, *maybe_tbl, acc_sh,
         ones_v, deg_sh) = refs
    else:
        (acc_out, gidx_v, didx_v, rows_v, *maybe_tbl, acc_sh) = refs
    c = lax.axis_index("c")
    s = lax.axis_index("s")
    wid = s * NC + c

    pltpu.sync_copy(gidx_hbm.at[wid], gidx_v)
    pltpu.sync_copy(didx_hbm.at[wid], didx_v)
    if staged:
        table_sh = maybe_tbl[0]
        tslc = trows // NS
        pltpu.sync_copy(table_hbm.at[pl.ds(s * tslc, tslc)],
                        table_sh.at[pl.ds(s * tslc, tslc)])
    else:
        table_sh = table_hbm

    zv = jnp.zeros((16,), jnp.float32)

    def zrow(i, _):
        for k in range(F // 16):
            rows_v[i, pl.ds(k * 16, 16)] = zv
        return 0

    lax.fori_loop(0, CH, zrow, 0)
    base = s * RPT
    for off in range(0, RPT, CH):
        ln = min(CH, RPT - off)
        pltpu.sync_copy(rows_v.at[pl.ds(0, ln)], acc_sh.at[pl.ds(base + off, ln)])

    if with_deg:
        def zrow2(i, _):
            ones_v[i, pl.ds(0, 16)] = zv
            return 0

        lax.fori_loop(0, CH, zrow2, 0)
        for off in range(0, RPT, CH):
            ln = min(CH, RPT - off)
            pltpu.sync_copy(ones_v.at[pl.ds(0, ln)], deg_sh.at[pl.ds(base + off, ln)])
        ov = jnp.ones((16,), jnp.float32)

        def orow(i, _):
            ones_v[i, pl.ds(0, 16)] = ov
            return 0

        lax.fori_loop(0, CH, orow, 0)

    plsc.subcore_barrier()

    def chunk(j, _):
        pltpu.sync_copy(table_sh.at[gidx_v.at[j]], rows_v)
        pltpu.sync_copy(rows_v, acc_sh.at[didx_v.at[j]], add=True)
        if with_deg:
            pltpu.sync_copy(ones_v, deg_sh.at[didx_v.at[j]], add=True)
        return 0

    lax.fori_loop(0, NCHUNK, chunk, 0)

    plsc.subcore_barrier()

    pltpu.sync_copy(acc_sh.at[pl.ds(base, RPT)], acc_out.at[c, pl.ds(base, RPT)])
    if with_deg:
        pltpu.sync_copy(deg_sh.at[pl.ds(base, RPT)], deg_out.at[c, pl.ds(base, RPT)])


def _make_edge_pass(F, trows, with_deg, staged):
    mesh = plsc.VectorSubcoreMesh(core_axis_name="c", subcore_axis_name="s")
    out_type = [jax.ShapeDtypeStruct((NC, NPAD, F), jnp.float32)]
    scratch = [
        pltpu.VMEM((NCHUNK, CH), jnp.int32),
        pltpu.VMEM((NCHUNK, CH), jnp.int32),
        pltpu.VMEM((CH, F), jnp.float32),
    ]
    if staged:
        scratch.append(pltpu.VMEM_SHARED((trows, F), jnp.float32))
    scratch.append(pltpu.VMEM_SHARED((NPAD, F), jnp.float32))
    if with_deg:
        out_type.append(jax.ShapeDtypeStruct((NC, NPAD, DEGW), jnp.float32))
        scratch += [
            pltpu.VMEM((CH, DEGW), jnp.float32),
            pltpu.VMEM_SHARED((NPAD, DEGW), jnp.float32),
        ]
    return pl.kernel(
        functools.partial(_edge_pass_body, with_deg, F, trows, staged),
        out_type=out_type,
        mesh=mesh,
        scratch_types=scratch,
        compiler_params=pltpu.CompilerParams(use_tc_tiling_on_sc=False),
    )


_edge_pass_rgcn = _make_edge_pass(HID, R * N, True, False)
_edge_pass_gcn = _make_edge_pass(F2, N, False, True)


def _tc_a_body(x_ref, wrel_ref, wself_ref, brg_ref, hrel_ref, selfp_ref):
    xb = x_ref[...]
    for r in range(R):
        hrel_ref[r] = jnp.dot(xb, wrel_ref[r], preferred_element_type=jnp.float32)
    selfp_ref[...] = (
        jnp.dot(xb, wself_ref[...], preferred_element_type=jnp.float32)
        + brg_ref[...]
    )


_tc_a = pl.pallas_call(
    _tc_a_body,
    grid=(NBLK,),
    in_specs=[
        pl.BlockSpec((NB, D), lambda i: (i, 0)),
        pl.BlockSpec((R, D, HID), lambda i: (0, 0, 0)),
        pl.BlockSpec((D, HID), lambda i: (0, 0)),
        pl.BlockSpec((1, HID), lambda i: (0, 0)),
    ],
    out_specs=[
        pl.BlockSpec((R, NB, HID), lambda i: (0, i, 0)),
        pl.BlockSpec((NB, HID), lambda i: (i, 0)),
    ],
    out_shape=[
        jax.ShapeDtypeStruct((R, N, HID), jnp.float32),
        jax.ShapeDtypeStruct((N, HID), jnp.float32),
    ],
)


def _tc_c_body(acc_ref, selfp_ref, degp_ref, w1_ref, g1_ref, dinv_ref):
    h0 = jnp.maximum(acc_ref[0] + acc_ref[1] + selfp_ref[...], 0.0)
    deg = degp_ref[0] + degp_ref[1] + 1.0
    dinv = lax.rsqrt(deg)
    dinv_ref[...] = dinv
    g1_ref[...] = dinv[:, 0:1] * jnp.dot(
        h0, w1_ref[...], preferred_element_type=jnp.float32)


_tc_c = pl.pallas_call(
    _tc_c_body,
    grid=(NBLK,),
    in_specs=[
        pl.BlockSpec((NC, NB, HID), lambda i: (0, i, 0)),
        pl.BlockSpec((NB, HID), lambda i: (i, 0)),
        pl.BlockSpec((NC, NB, DEGW), lambda i: (0, i, 0)),
        pl.BlockSpec((HID, F2), lambda i: (0, 0)),
    ],
    out_specs=[
        pl.BlockSpec((NB, F2), lambda i: (i, 0)),
        pl.BlockSpec((NB, DEGW), lambda i: (i, 0)),
    ],
    out_shape=[
        jax.ShapeDtypeStruct((N, F2), jnp.float32),
        jax.ShapeDtypeStruct((N, DEGW), jnp.float32),
    ],
)


def _tc_e_body(acc_ref, g1_ref, dinv_ref, w3_ref, b1_ref, g2_ref):
    dinv1 = dinv_ref[:, 0:1]
    out1 = jnp.maximum(
        dinv1 * (acc_ref[0] + acc_ref[1] + g1_ref[...]) + b1_ref[...], 0.0)
    g2_ref[...] = dinv1 * jnp.dot(
        out1, w3_ref[...], preferred_element_type=jnp.float32)


_tc_e = pl.pallas_call(
    _tc_e_body,
    grid=(NBLK,),
    in_specs=[
        pl.BlockSpec((NC, NB, F2), lambda i: (0, i, 0)),
        pl.BlockSpec((NB, F2), lambda i: (i, 0)),
        pl.BlockSpec((NB, DEGW), lambda i: (i, 0)),
        pl.BlockSpec((F2, F2), lambda i: (0, 0)),
        pl.BlockSpec((1, F2), lambda i: (0, 0)),
    ],
    out_specs=pl.BlockSpec((NB, F2), lambda i: (i, 0)),
    out_shape=jax.ShapeDtypeStruct((N, F2), jnp.float32),
)


def _tc_g_body(acc_ref, g2_ref, dinv_ref, batch_ref, b3_ref, wl1_ref, bl1_ref,
               wl2_ref, bl2_ref, out_ref, pool_ref, cnt_ref):
    i = pl.program_id(0)

    @pl.when(i == 0)
    def _init():
        pool_ref[...] = jnp.zeros_like(pool_ref)
        cnt_ref[...] = jnp.zeros_like(cnt_ref)

    dinv1 = dinv_ref[:, 0:1]
    h2 = dinv1 * (acc_ref[0] + acc_ref[1] + g2_ref[...]) + b3_ref[...]
    onehot = (batch_ref[...] == lax.broadcasted_iota(
        jnp.int32, (NB, G), 1)).astype(jnp.float32)
    dn = (((0,), (0,)), ((), ()))
    pool_ref[...] += lax.dot_general(
        onehot, h2, dn, preferred_element_type=jnp.float32)
    cnt_ref[...] += lax.dot_general(
        onehot, jnp.ones((NB, 1), jnp.float32), dn,
        preferred_element_type=jnp.float32)

    @pl.when(i == NBLK - 1)
    def _fin():
        pooled = pool_ref[...] / jnp.maximum(cnt_ref[...], 1.0)
        o1 = jnp.dot(pooled, wl1_ref[...],
                     preferred_element_type=jnp.float32) + bl1_ref[...]
        o2 = jnp.dot(o1, wl2_ref[...],
                     preferred_element_type=jnp.float32) + bl2_ref[...]
        m = jnp.max(o2, axis=-1, keepdims=True)
        ex = jnp.exp(o2 - m)
        out_ref[...] = ex / jnp.sum(ex, axis=-1, keepdims=True)


_tc_g = pl.pallas_call(
    _tc_g_body,
    grid=(NBLK,),
    in_specs=[
        pl.BlockSpec((NC, NB, F2), lambda i: (0, i, 0)),
        pl.BlockSpec((NB, F2), lambda i: (i, 0)),
        pl.BlockSpec((NB, DEGW), lambda i: (i, 0)),
        pl.BlockSpec((NB, 1), lambda i: (i, 0)),
        pl.BlockSpec((1, F2), lambda i: (0, 0)),
        pl.BlockSpec((F2, 32), lambda i: (0, 0)),
        pl.BlockSpec((1, 32), lambda i: (0, 0)),
        pl.BlockSpec((32, OUT), lambda i: (0, 0)),
        pl.BlockSpec((1, OUT), lambda i: (0, 0)),
    ],
    out_specs=pl.BlockSpec((G, OUT), lambda i: (0, 0)),
    out_shape=jax.ShapeDtypeStruct((G, OUT), jnp.float32),
    scratch_shapes=[
        pltpu.VMEM((G, F2), jnp.float32),
        pltpu.VMEM((G, 1), jnp.float32),
    ],
)


def kernel(x, edge_index, edge_attr, batch, W_rel, W_self, b_rgcn,
           W1, b1, W3, b3, Wl1, bl1, Wl2, bl2):
    src = edge_index[0]
    dst = edge_index[1]
    pad = E_PAD - E
    zpad = jnp.zeros((pad,), jnp.int32)
    gidx1 = jnp.concatenate([edge_attr * N + src, zpad]).reshape(NW, NCHUNK, CH)
    src_p = jnp.concatenate([src, zpad]).reshape(NW, NCHUNK, CH)
    didx = jnp.concatenate([dst, jnp.full((pad,), N, jnp.int32)]
                           ).reshape(NW, NCHUNK, CH)

    hrel, selfp = _tc_a(x, W_rel, W_self, b_rgcn.reshape(1, HID))
    acc0, degp = _edge_pass_rgcn(hrel.reshape(R * N, HID), gidx1, didx)
    g1, dinv = _tc_c(acc0, selfp, degp, W1)
    acc1, = _edge_pass_gcn(g1, src_p, didx)
    g2 = _tc_e(acc1, g1, dinv, W3, b1.reshape(1, F2))
    acc2, = _edge_pass_gcn(g2, src_p, didx)
    return _tc_g(acc2, g2, dinv, batch.reshape(N, 1), b3.reshape(1, F2),
                 Wl1, bl1.reshape(1, 32), Wl2, bl2.reshape(1, OUT))


# speedup vs baseline: 1.7650x; 1.1808x over previous
import functools

import jax
import jax.numpy as jnp
from jax import lax
from jax.experimental import pallas as pl
from jax.experimental.pallas import tpu as pltpu
from jax.experimental.pallas import tpu_sc as plsc

N, E, D, R, G = 10000, 320000, 128, 4, 64
HID, F2, OUT = 32, 64, 10

NC, NS = 2, 16
NW = NC * NS
CH = 128
EPT = E // NW
NBUF = 8
NCHUNK = 80
EPT_PAD = NCHUNK * CH
E_PAD = EPT_PAD * NW
NPAD = 10112
RPT = NPAD // NS
DEGW = 8
NB = 1000
NBLK = N // NB


def _edge_pass_body(with_deg, F, trows, staged, *args):
    if with_deg:
        (table_hbm, gidx_hbm, didx_hbm, zacc_hbm, zdeg_hbm, ones_hbm,
         acc_out, deg_out, gidx_v, didx_v, rows_v, *maybe_tbl, acc_sh,
         ones_v, deg_sh) = args
    else:
        (table_hbm, gidx_hbm, didx_hbm, zacc_hbm,
         acc_out, gidx_v, didx_v, rows_v, *maybe_tbl, acc_sh) = args
    c = lax.axis_index("c")
    s = lax.axis_index("s")
    wid = s * NC + c

    pltpu.sync_copy(gidx_hbm.at[wid], gidx_v)
    pltpu.sync_copy(didx_hbm.at[wid], didx_v)
    if staged:
        table_sh = maybe_tbl[0]
        tslc = trows // NS
        pltpu.sync_copy(table_hbm.at[pl.ds(s * tslc, tslc)],
                        table_sh.at[pl.ds(s * tslc, tslc)])
    else:
        table_sh = table_hbm

    base = s * RPT
    pltpu.sync_copy(zacc_hbm, acc_sh.at[pl.ds(base, RPT)])
    if with_deg:
        pltpu.sync_copy(zdeg_hbm, deg_sh.at[pl.ds(base, RPT)])
        pltpu.sync_copy(ones_hbm, ones_v)

    plsc.subcore_barrier()

    def chunk(j, _):
        pltpu.sync_copy(table_sh.at[gidx_v.at[j]], rows_v)
        pltpu.sync_copy(rows_v, acc_sh.at[didx_v.at[j]], add=True)
        if with_deg:
            pltpu.sync_copy(ones_v, deg_sh.at[didx_v.at[j]], add=True)
        return 0

    lax.fori_loop(0, NCHUNK, chunk, 0)

    plsc.subcore_barrier()

    pltpu.sync_copy(acc_sh.at[pl.ds(base, RPT)], acc_out.at[c, pl.ds(base, RPT)])
    if with_deg:
        pltpu.sync_copy(deg_sh.at[pl.ds(base, RPT)], deg_out.at[c, pl.ds(base, RPT)])


def _make_edge_pass(F, trows, with_deg, staged):
    mesh = plsc.VectorSubcoreMesh(core_axis_name="c", subcore_axis_name="s")
    out_type = [jax.ShapeDtypeStruct((NC, NPAD, F), jnp.float32)]
    scratch = [
        pltpu.VMEM((NCHUNK, CH), jnp.int32),
        pltpu.VMEM((NCHUNK, CH), jnp.int32),
        pltpu.VMEM((CH, F), jnp.float32),
    ]
    if staged:
        scratch.append(pltpu.VMEM_SHARED((trows, F), jnp.float32))
    scratch.append(pltpu.VMEM_SHARED((NPAD, F), jnp.float32))
    if with_deg:
        out_type.append(jax.ShapeDtypeStruct((NC, NPAD, DEGW), jnp.float32))
        scratch += [
            pltpu.VMEM((CH, DEGW), jnp.float32),
            pltpu.VMEM_SHARED((NPAD, DEGW), jnp.float32),
        ]
    return pl.kernel(
        functools.partial(_edge_pass_body, with_deg, F, trows, staged),
        out_type=out_type,
        mesh=mesh,
        scratch_types=scratch,
        compiler_params=pltpu.CompilerParams(use_tc_tiling_on_sc=False),
    )


_edge_pass_rgcn = _make_edge_pass(HID, R * N, True, True)
_edge_pass_gcn = _make_edge_pass(F2, N, False, True)


def _tc_a_body(x_ref, wrel_ref, wself_ref, brg_ref, hrel_ref, selfp_ref):
    xb = x_ref[...]
    for r in range(R):
        hrel_ref[r] = jnp.dot(xb, wrel_ref[r], preferred_element_type=jnp.float32)
    selfp_ref[...] = (
        jnp.dot(xb, wself_ref[...], preferred_element_type=jnp.float32)
        + brg_ref[...]
    )


_tc_a = pl.pallas_call(
    _tc_a_body,
    grid=(NBLK,),
    in_specs=[
        pl.BlockSpec((NB, D), lambda i: (i, 0)),
        pl.BlockSpec((R, D, HID), lambda i: (0, 0, 0)),
        pl.BlockSpec((D, HID), lambda i: (0, 0)),
        pl.BlockSpec((1, HID), lambda i: (0, 0)),
    ],
    out_specs=[
        pl.BlockSpec((R, NB, HID), lambda i: (0, i, 0)),
        pl.BlockSpec((NB, HID), lambda i: (i, 0)),
    ],
    out_shape=[
        jax.ShapeDtypeStruct((R, N, HID), jnp.float32),
        jax.ShapeDtypeStruct((N, HID), jnp.float32),
    ],
)


def _tc_c_body(acc_ref, selfp_ref, degp_ref, w1_ref, g1_ref, dinv_ref):
    h0 = jnp.maximum(acc_ref[0] + acc_ref[1] + selfp_ref[...], 0.0)
    deg = degp_ref[0] + degp_ref[1] + 1.0
    dinv = lax.rsqrt(deg)
    dinv_ref[...] = dinv
    g1_ref[...] = dinv[:, 0:1] * jnp.dot(
        h0, w1_ref[...], preferred_element_type=jnp.float32)


_tc_c = pl.pallas_call(
    _tc_c_body,
    grid=(NBLK,),
    in_specs=[
        pl.BlockSpec((NC, NB, HID), lambda i: (0, i, 0)),
        pl.BlockSpec((NB, HID), lambda i: (i, 0)),
        pl.BlockSpec((NC, NB, DEGW), lambda i: (0, i, 0)),
        pl.BlockSpec((HID, F2), lambda i: (0, 0)),
    ],
    out_specs=[
        pl.BlockSpec((NB, F2), lambda i: (i, 0)),
        pl.BlockSpec((NB, DEGW), lambda i: (i, 0)),
    ],
    out_shape=[
        jax.ShapeDtypeStruct((N, F2), jnp.float32),
        jax.ShapeDtypeStruct((N, DEGW), jnp.float32),
    ],
)


def _tc_e_body(acc_ref, g1_ref, dinv_ref, w3_ref, b1_ref, g2_ref):
    dinv1 = dinv_ref[:, 0:1]
    out1 = jnp.maximum(
        dinv1 * (acc_ref[0] + acc_ref[1] + g1_ref[...]) + b1_ref[...], 0.0)
    g2_ref[...] = dinv1 * jnp.dot(
        out1, w3_ref[...], preferred_element_type=jnp.float32)


_tc_e = pl.pallas_call(
    _tc_e_body,
    grid=(NBLK,),
    in_specs=[
        pl.BlockSpec((NC, NB, F2), lambda i: (0, i, 0)),
        pl.BlockSpec((NB, F2), lambda i: (i, 0)),
        pl.BlockSpec((NB, DEGW), lambda i: (i, 0)),
        pl.BlockSpec((F2, F2), lambda i: (0, 0)),
        pl.BlockSpec((1, F2), lambda i: (0, 0)),
    ],
    out_specs=pl.BlockSpec((NB, F2), lambda i: (i, 0)),
    out_shape=jax.ShapeDtypeStruct((N, F2), jnp.float32),
)


def _tc_g_body(acc_ref, g2_ref, dinv_ref, batch_ref, b3_ref, wl1_ref, bl1_ref,
               wl2_ref, bl2_ref, out_ref, pool_ref, cnt_ref):
    i = pl.program_id(0)

    @pl.when(i == 0)
    def _init():
        pool_ref[...] = jnp.zeros_like(pool_ref)
        cnt_ref[...] = jnp.zeros_like(cnt_ref)

    dinv1 = dinv_ref[:, 0:1]
    h2 = dinv1 * (acc_ref[0] + acc_ref[1] + g2_ref[...]) + b3_ref[...]
    onehot = (batch_ref[...] == lax.broadcasted_iota(
        jnp.int32, (NB, G), 1)).astype(jnp.float32)
    dn = (((0,), (0,)), ((), ()))
    pool_ref[...] += lax.dot_general(
        onehot, h2, dn, preferred_element_type=jnp.float32)
    cnt_ref[...] += lax.dot_general(
        onehot, jnp.ones((NB, 1), jnp.float32), dn,
        preferred_element_type=jnp.float32)

    @pl.when(i == NBLK - 1)
    def _fin():
        pooled = pool_ref[...] / jnp.maximum(cnt_ref[...], 1.0)
        o1 = jnp.dot(pooled, wl1_ref[...],
                     preferred_element_type=jnp.float32) + bl1_ref[...]
        o2 = jnp.dot(o1, wl2_ref[...],
                     preferred_element_type=jnp.float32) + bl2_ref[...]
        m = jnp.max(o2, axis=-1, keepdims=True)
        ex = jnp.exp(o2 - m)
        out_ref[...] = ex / jnp.sum(ex, axis=-1, keepdims=True)


_tc_g = pl.pallas_call(
    _tc_g_body,
    grid=(NBLK,),
    in_specs=[
        pl.BlockSpec((NC, NB, F2), lambda i: (0, i, 0)),
        pl.BlockSpec((NB, F2), lambda i: (i, 0)),
        pl.BlockSpec((NB, DEGW), lambda i: (i, 0)),
        pl.BlockSpec((NB, 1), lambda i: (i, 0)),
        pl.BlockSpec((1, F2), lambda i: (0, 0)),
        pl.BlockSpec((F2, 32), lambda i: (0, 0)),
        pl.BlockSpec((1, 32), lambda i: (0, 0)),
        pl.BlockSpec((32, OUT), lambda i: (0, 0)),
        pl.BlockSpec((1, OUT), lambda i: (0, 0)),
    ],
    out_specs=pl.BlockSpec((G, OUT), lambda i: (0, 0)),
    out_shape=jax.ShapeDtypeStruct((G, OUT), jnp.float32),
    scratch_shapes=[
        pltpu.VMEM((G, F2), jnp.float32),
        pltpu.VMEM((G, 1), jnp.float32),
    ],
)


def kernel(x, edge_index, edge_attr, batch, W_rel, W_self, b_rgcn,
           W1, b1, W3, b3, Wl1, bl1, Wl2, bl2):
    src = edge_index[0]
    dst = edge_index[1]
    pad = E_PAD - E
    zpad = jnp.zeros((pad,), jnp.int32)
    gidx1 = jnp.concatenate([edge_attr * N + src, zpad]).reshape(NW, NCHUNK, CH)
    src_p = jnp.concatenate([src, zpad]).reshape(NW, NCHUNK, CH)
    didx = jnp.concatenate([dst, jnp.full((pad,), N, jnp.int32)]
                           ).reshape(NW, NCHUNK, CH)

    zacc1 = jnp.zeros((RPT, HID), jnp.float32)
    zacc2 = jnp.zeros((RPT, F2), jnp.float32)
    zdeg = jnp.zeros((RPT, DEGW), jnp.float32)
    ones_deg = jnp.ones((CH, DEGW), jnp.float32)

    hrel, selfp = _tc_a(x, W_rel, W_self, b_rgcn.reshape(1, HID))
    acc0, degp = _edge_pass_rgcn(hrel.reshape(R * N, HID), gidx1, didx,
                                 zacc1, zdeg, ones_deg)
    g1, dinv = _tc_c(acc0, selfp, degp, W1)
    acc1, = _edge_pass_gcn(g1, src_p, didx, zacc2)
    g2 = _tc_e(acc1, g1, dinv, W3, b1.reshape(1, F2))
    acc2, = _edge_pass_gcn(g2, src_p, didx, zacc2)
    return _tc_g(acc2, g2, dinv, batch.reshape(N, 1), b3.reshape(1, F2),
                 Wl1, bl1.reshape(1, 32), Wl2, bl2.reshape(1, OUT))


# speedup vs baseline: 1.7873x; 1.0126x over previous
import functools

import jax
import jax.numpy as jnp
from jax import lax
from jax.experimental import pallas as pl
from jax.experimental.pallas import tpu as pltpu
from jax.experimental.pallas import tpu_sc as plsc

N, E, D, R, G = 10000, 320000, 128, 4, 64
HID, F2, OUT = 32, 64, 10

NC, NS = 2, 16
NW = NC * NS
CH = 128
EPT = E // NW
NBUF = 8
NCHUNK = 80
EPT_PAD = NCHUNK * CH
E_PAD = EPT_PAD * NW
NPAD = 10112
RPT = NPAD // NS
DEGW = 8
NB = 10000
NBLK = N // NB


def _edge_pass_body(with_deg, F, trows, staged, *args):
    if with_deg:
        (table_hbm, gidx_hbm, didx_hbm, zacc_hbm, zdeg_hbm, ones_hbm,
         acc_out, deg_out, gidx_v, didx_v, rows_v, *maybe_tbl, acc_sh,
         ones_v, deg_sh) = args
    else:
        (table_hbm, gidx_hbm, didx_hbm, zacc_hbm,
         acc_out, gidx_v, didx_v, rows_v, *maybe_tbl, acc_sh) = args
    c = lax.axis_index("c")
    s = lax.axis_index("s")
    wid = s * NC + c

    pltpu.sync_copy(gidx_hbm.at[wid], gidx_v)
    pltpu.sync_copy(didx_hbm.at[wid], didx_v)
    if staged:
        table_sh = maybe_tbl[0]
        tslc = trows // NS
        pltpu.sync_copy(table_hbm.at[pl.ds(s * tslc, tslc)],
                        table_sh.at[pl.ds(s * tslc, tslc)])
    else:
        table_sh = table_hbm

    base = s * RPT
    pltpu.sync_copy(zacc_hbm, acc_sh.at[pl.ds(base, RPT)])
    if with_deg:
        pltpu.sync_copy(zdeg_hbm, deg_sh.at[pl.ds(base, RPT)])
        pltpu.sync_copy(ones_hbm, ones_v)

    plsc.subcore_barrier()

    def chunk(j, _):
        pltpu.sync_copy(table_sh.at[gidx_v.at[j]], rows_v)
        pltpu.sync_copy(rows_v, acc_sh.at[didx_v.at[j]], add=True)
        if with_deg:
            pltpu.sync_copy(ones_v, deg_sh.at[didx_v.at[j]], add=True)
        return 0

    lax.fori_loop(0, NCHUNK, chunk, 0)

    plsc.subcore_barrier()

    pltpu.sync_copy(acc_sh.at[pl.ds(base, RPT)], acc_out.at[c, pl.ds(base, RPT)])
    if with_deg:
        pltpu.sync_copy(deg_sh.at[pl.ds(base, RPT)], deg_out.at[c, pl.ds(base, RPT)])


def _make_edge_pass(F, trows, with_deg, staged):
    mesh = plsc.VectorSubcoreMesh(core_axis_name="c", subcore_axis_name="s")
    out_type = [jax.ShapeDtypeStruct((NC, NPAD, F), jnp.float32)]
    scratch = [
        pltpu.VMEM((NCHUNK, CH), jnp.int32),
        pltpu.VMEM((NCHUNK, CH), jnp.int32),
        pltpu.VMEM((CH, F), jnp.float32),
    ]
    if staged:
        scratch.append(pltpu.VMEM_SHARED((trows, F), jnp.float32))
    scratch.append(pltpu.VMEM_SHARED((NPAD, F), jnp.float32))
    if with_deg:
        out_type.append(jax.ShapeDtypeStruct((NC, NPAD, DEGW), jnp.float32))
        scratch += [
            pltpu.VMEM((CH, DEGW), jnp.float32),
            pltpu.VMEM_SHARED((NPAD, DEGW), jnp.float32),
        ]
    return pl.kernel(
        functools.partial(_edge_pass_body, with_deg, F, trows, staged),
        out_type=out_type,
        mesh=mesh,
        scratch_types=scratch,
        compiler_params=pltpu.CompilerParams(use_tc_tiling_on_sc=False),
    )


_edge_pass_rgcn = _make_edge_pass(HID, R * N, True, True)
_edge_pass_gcn = _make_edge_pass(F2, N, False, True)


def _tc_a_body(x_ref, wrel_ref, wself_ref, brg_ref, hrel_ref, selfp_ref):
    xb = x_ref[...]
    for r in range(R):
        hrel_ref[r] = jnp.dot(xb, wrel_ref[r], preferred_element_type=jnp.float32)
    selfp_ref[...] = (
        jnp.dot(xb, wself_ref[...], preferred_element_type=jnp.float32)
        + brg_ref[...]
    )


_tc_a = pl.pallas_call(
    _tc_a_body,
    grid=(NBLK,),
    in_specs=[
        pl.BlockSpec((NB, D), lambda i: (i, 0)),
        pl.BlockSpec((R, D, HID), lambda i: (0, 0, 0)),
        pl.BlockSpec((D, HID), lambda i: (0, 0)),
        pl.BlockSpec((1, HID), lambda i: (0, 0)),
    ],
    out_specs=[
        pl.BlockSpec((R, NB, HID), lambda i: (0, i, 0)),
        pl.BlockSpec((NB, HID), lambda i: (i, 0)),
    ],
    out_shape=[
        jax.ShapeDtypeStruct((R, N, HID), jnp.float32),
        jax.ShapeDtypeStruct((N, HID), jnp.float32),
    ],
)


def _tc_c_body(acc_ref, selfp_ref, degp_ref, w1_ref, g1_ref, dinv_ref):
    h0 = jnp.maximum(acc_ref[0] + acc_ref[1] + selfp_ref[...], 0.0)
    deg = degp_ref[0] + degp_ref[1] + 1.0
    dinv = lax.rsqrt(deg)
    dinv_ref[...] = dinv
    g1_ref[...] = dinv[:, 0:1] * jnp.dot(
        h0, w1_ref[...], preferred_element_type=jnp.float32)


_tc_c = pl.pallas_call(
    _tc_c_body,
    grid=(NBLK,),
    in_specs=[
        pl.BlockSpec((NC, NB, HID), lambda i: (0, i, 0)),
        pl.BlockSpec((NB, HID), lambda i: (i, 0)),
        pl.BlockSpec((NC, NB, DEGW), lambda i: (0, i, 0)),
        pl.BlockSpec((HID, F2), lambda i: (0, 0)),
    ],
    out_specs=[
        pl.BlockSpec((NB, F2), lambda i: (i, 0)),
        pl.BlockSpec((NB, DEGW), lambda i: (i, 0)),
    ],
    out_shape=[
        jax.ShapeDtypeStruct((N, F2), jnp.float32),
        jax.ShapeDtypeStruct((N, DEGW), jnp.float32),
    ],
)


def _tc_e_body(acc_ref, g1_ref, dinv_ref, w3_ref, b1_ref, g2_ref):
    dinv1 = dinv_ref[:, 0:1]
    out1 = jnp.maximum(
        dinv1 * (acc_ref[0] + acc_ref[1] + g1_ref[...]) + b1_ref[...], 0.0)
    g2_ref[...] = dinv1 * jnp.dot(
        out1, w3_ref[...], preferred_element_type=jnp.float32)


_tc_e = pl.pallas_call(
    _tc_e_body,
    grid=(NBLK,),
    in_specs=[
        pl.BlockSpec((NC, NB, F2), lambda i: (0, i, 0)),
        pl.BlockSpec((NB, F2), lambda i: (i, 0)),
        pl.BlockSpec((NB, DEGW), lambda i: (i, 0)),
        pl.BlockSpec((F2, F2), lambda i: (0, 0)),
        pl.BlockSpec((1, F2), lambda i: (0, 0)),
    ],
    out_specs=pl.BlockSpec((NB, F2), lambda i: (i, 0)),
    out_shape=jax.ShapeDtypeStruct((N, F2), jnp.float32),
)


def _tc_g_body(acc_ref, g2_ref, dinv_ref, batch_ref, b3_ref, wl1_ref, bl1_ref,
               wl2_ref, bl2_ref, out_ref, pool_ref, cnt_ref):
    i = pl.program_id(0)

    @pl.when(i == 0)
    def _init():
        pool_ref[...] = jnp.zeros_like(pool_ref)
        cnt_ref[...] = jnp.zeros_like(cnt_ref)

    dinv1 = dinv_ref[:, 0:1]
    h2 = dinv1 * (acc_ref[0] + acc_ref[1] + g2_ref[...]) + b3_ref[...]
    onehot = (batch_ref[...] == lax.broadcasted_iota(
        jnp.int32, (NB, G), 1)).astype(jnp.float32)
    dn = (((0,), (0,)), ((), ()))
    pool_ref[...] += lax.dot_general(
        onehot, h2, dn, preferred_element_type=jnp.float32)
    cnt_ref[...] += lax.dot_general(
        onehot, jnp.ones((NB, 1), jnp.float32), dn,
        preferred_element_type=jnp.float32)

    @pl.when(i == NBLK - 1)
    def _fin():
        pooled = pool_ref[...] / jnp.maximum(cnt_ref[...], 1.0)
        o1 = jnp.dot(pooled, wl1_ref[...],
                     preferred_element_type=jnp.float32) + bl1_ref[...]
        o2 = jnp.dot(o1, wl2_ref[...],
                     preferred_element_type=jnp.float32) + bl2_ref[...]
        m = jnp.max(o2, axis=-1, keepdims=True)
        ex = jnp.exp(o2 - m)
        out_ref[...] = ex / jnp.sum(ex, axis=-1, keepdims=True)


_tc_g = pl.pallas_call(
    _tc_g_body,
    grid=(NBLK,),
    in_specs=[
        pl.BlockSpec((NC, NB, F2), lambda i: (0, i, 0)),
        pl.BlockSpec((NB, F2), lambda i: (i, 0)),
        pl.BlockSpec((NB, DEGW), lambda i: (i, 0)),
        pl.BlockSpec((NB, 1), lambda i: (i, 0)),
        pl.BlockSpec((1, F2), lambda i: (0, 0)),
        pl.BlockSpec((F2, 32), lambda i: (0, 0)),
        pl.BlockSpec((1, 32), lambda i: (0, 0)),
        pl.BlockSpec((32, OUT), lambda i: (0, 0)),
        pl.BlockSpec((1, OUT), lambda i: (0, 0)),
    ],
    out_specs=pl.BlockSpec((G, OUT), lambda i: (0, 0)),
    out_shape=jax.ShapeDtypeStruct((G, OUT), jnp.float32),
    scratch_shapes=[
        pltpu.VMEM((G, F2), jnp.float32),
        pltpu.VMEM((G, 1), jnp.float32),
    ],
)


def kernel(x, edge_index, edge_attr, batch, W_rel, W_self, b_rgcn,
           W1, b1, W3, b3, Wl1, bl1, Wl2, bl2):
    src = edge_index[0]
    dst = edge_index[1]
    pad = E_PAD - E
    zpad = jnp.zeros((pad,), jnp.int32)
    gidx1 = jnp.concatenate([edge_attr * N + src, zpad]).reshape(NW, NCHUNK, CH)
    src_p = jnp.concatenate([src, zpad]).reshape(NW, NCHUNK, CH)
    didx = jnp.concatenate([dst, jnp.full((pad,), N, jnp.int32)]
                           ).reshape(NW, NCHUNK, CH)

    zacc1 = jnp.zeros((RPT, HID), jnp.float32)
    zacc2 = jnp.zeros((RPT, F2), jnp.float32)
    zdeg = jnp.zeros((RPT, DEGW), jnp.float32)
    ones_deg = jnp.ones((CH, DEGW), jnp.float32)

    hrel, selfp = _tc_a(x, W_rel, W_self, b_rgcn.reshape(1, HID))
    acc0, degp = _edge_pass_rgcn(hrel.reshape(R * N, HID), gidx1, didx,
                                 zacc1, zdeg, ones_deg)
    g1, dinv = _tc_c(acc0, selfp, degp, W1)
    acc1, = _edge_pass_gcn(g1, src_p, didx, zacc2)
    g2 = _tc_e(acc1, g1, dinv, W3, b1.reshape(1, F2))
    acc2, = _edge_pass_gcn(g2, src_p, didx, zacc2)
    return _tc_g(acc2, g2, dinv, batch.reshape(N, 1), b3.reshape(1, F2),
                 Wl1, bl1.reshape(1, 32), Wl2, bl2.reshape(1, OUT))


# speedup vs baseline: 2.0726x; 1.1596x over previous
import functools

import jax
import jax.numpy as jnp
from jax import lax
from jax.experimental import pallas as pl
from jax.experimental.pallas import tpu as pltpu
from jax.experimental.pallas import tpu_sc as plsc

N, E, D, R, G = 10000, 320000, 128, 4, 64
HID, F2, OUT = 32, 64, 10

NC, NS = 2, 16
NW = NC * NS
CH = 128
EPT = E // NW
NBUF = 8
NCHUNK = 80
EPT_PAD = NCHUNK * CH
E_PAD = EPT_PAD * NW
NPAD = 10112
RPT = NPAD // NS
DEGW = 8
NB = 10000
NBLK = N // NB


def _edge_pass_body(with_deg, F, trows, staged, *args):
    if with_deg:
        (table_hbm, gidx_hbm, didx_hbm, zacc_hbm, zdeg_hbm, ones_hbm,
         acc_out, deg_out, gidx_v, didx_v, rows_v,
         *maybe_tbl, acc_sh, ones_v, deg_sh) = args
    else:
        (table_hbm, gidx_hbm, didx_hbm, zacc_hbm,
         acc_out, gidx_v, didx_v, rows_v, rows_w, gsa, gsb,
         *maybe_tbl, acc_sh) = args
    c = lax.axis_index("c")
    s = lax.axis_index("s")
    wid = s * NC + c

    pltpu.sync_copy(gidx_hbm.at[wid], gidx_v)
    pltpu.sync_copy(didx_hbm.at[wid], didx_v)
    if staged:
        table_sh = maybe_tbl[0]
        tslc = trows // NS
        pltpu.sync_copy(table_hbm.at[pl.ds(s * tslc, tslc)],
                        table_sh.at[pl.ds(s * tslc, tslc)])
    else:
        table_sh = table_hbm

    base = s * RPT
    pltpu.sync_copy(zacc_hbm, acc_sh.at[pl.ds(base, RPT)])
    if with_deg:
        pltpu.sync_copy(zdeg_hbm, deg_sh.at[pl.ds(base, RPT)])
        pltpu.sync_copy(ones_hbm, ones_v)

    plsc.subcore_barrier()

    if with_deg:
        def chunk(j, _):
            pltpu.sync_copy(table_sh.at[gidx_v.at[j]], rows_v)
            pltpu.sync_copy(rows_v, acc_sh.at[didx_v.at[j]], add=True)
            pltpu.sync_copy(ones_v, deg_sh.at[didx_v.at[j]], add=True)
            return 0

        lax.fori_loop(0, NCHUNK, chunk, 0)
    else:
        def g_issue(j, buf, sem):
            pltpu.async_copy(table_sh.at[gidx_v.at[j]], buf, sem)

        def g_wait(buf, sem):
            pltpu.make_async_copy(table_sh.at[gidx_v.at[0]], buf, sem).wait()

        def scat(j, buf):
            pltpu.sync_copy(buf, acc_sh.at[didx_v.at[j]], add=True)

        g_issue(0, rows_v, gsa)

        def pair(p, _):
            j = p * 2
            g_wait(rows_v, gsa)
            g_issue(jnp.minimum(j + 1, NCHUNK - 1), rows_w, gsb)
            scat(j, rows_v)
            g_wait(rows_w, gsb)
            g_issue(jnp.minimum(j + 2, NCHUNK - 1), rows_v, gsa)
            scat(j + 1, rows_w)
            return 0

        lax.fori_loop(0, NCHUNK // 2, pair, 0)
        g_wait(rows_v, gsa)

    plsc.subcore_barrier()

    pltpu.sync_copy(acc_sh.at[pl.ds(base, RPT)], acc_out.at[c, pl.ds(base, RPT)])
    if with_deg:
        pltpu.sync_copy(deg_sh.at[pl.ds(base, RPT)], deg_out.at[c, pl.ds(base, RPT)])


def _make_edge_pass(F, trows, with_deg, staged):
    mesh = plsc.VectorSubcoreMesh(core_axis_name="c", subcore_axis_name="s")
    out_type = [jax.ShapeDtypeStruct((NC, NPAD, F), jnp.float32)]
    scratch = [
        pltpu.VMEM((NCHUNK, CH), jnp.int32),
        pltpu.VMEM((NCHUNK, CH), jnp.int32),
        pltpu.VMEM((CH, F), jnp.float32),
    ]
    if not with_deg:
        scratch += [
            pltpu.VMEM((CH, F), jnp.float32),
            pltpu.SemaphoreType.DMA,
            pltpu.SemaphoreType.DMA,
        ]
    if staged:
        scratch.append(pltpu.VMEM_SHARED((trows, F), jnp.float32))
    scratch.append(pltpu.VMEM_SHARED((NPAD, F), jnp.float32))
    if with_deg:
        out_type.append(jax.ShapeDtypeStruct((NC, NPAD, DEGW), jnp.float32))
        scratch += [
            pltpu.VMEM((CH, DEGW), jnp.float32),
            pltpu.VMEM_SHARED((NPAD, DEGW), jnp.float32),
        ]
    return pl.kernel(
        functools.partial(_edge_pass_body, with_deg, F, trows, staged),
        out_type=out_type,
        mesh=mesh,
        scratch_types=scratch,
        compiler_params=pltpu.CompilerParams(use_tc_tiling_on_sc=False),
    )


_edge_pass_rgcn = _make_edge_pass(HID, R * N, True, True)
_edge_pass_gcn = _make_edge_pass(F2, N, False, True)


def _tc_a_body(x_ref, wrel_ref, wself_ref, brg_ref, hrel_ref, selfp_ref):
    xb = x_ref[...]
    for r in range(R):
        hrel_ref[r] = jnp.dot(xb, wrel_ref[r], preferred_element_type=jnp.float32)
    selfp_ref[...] = (
        jnp.dot(xb, wself_ref[...], preferred_element_type=jnp.float32)
        + brg_ref[...]
    )


_tc_a = pl.pallas_call(
    _tc_a_body,
    grid=(NBLK,),
    in_specs=[
        pl.BlockSpec((NB, D), lambda i: (i, 0)),
        pl.BlockSpec((R, D, HID), lambda i: (0, 0, 0)),
        pl.BlockSpec((D, HID), lambda i: (0, 0)),
        pl.BlockSpec((1, HID), lambda i: (0, 0)),
    ],
    out_specs=[
        pl.BlockSpec((R, NB, HID), lambda i: (0, i, 0)),
        pl.BlockSpec((NB, HID), lambda i: (i, 0)),
    ],
    out_shape=[
        jax.ShapeDtypeStruct((R, N, HID), jnp.float32),
        jax.ShapeDtypeStruct((N, HID), jnp.float32),
    ],
)


def _tc_c_body(acc_ref, selfp_ref, degp_ref, w1_ref, g1_ref, dinv_ref):
    h0 = jnp.maximum(acc_ref[0] + acc_ref[1] + selfp_ref[...], 0.0)
    deg = degp_ref[0] + degp_ref[1] + 1.0
    dinv = lax.rsqrt(deg)
    dinv_ref[...] = dinv
    g1_ref[...] = dinv[:, 0:1] * jnp.dot(
        h0, w1_ref[...], preferred_element_type=jnp.float32)


_tc_c = pl.pallas_call(
    _tc_c_body,
    grid=(NBLK,),
    in_specs=[
        pl.BlockSpec((NC, NB, HID), lambda i: (0, i, 0)),
        pl.BlockSpec((NB, HID), lambda i: (i, 0)),
        pl.BlockSpec((NC, NB, DEGW), lambda i: (0, i, 0)),
        pl.BlockSpec((HID, F2), lambda i: (0, 0)),
    ],
    out_specs=[
        pl.BlockSpec((NB, F2), lambda i: (i, 0)),
        pl.BlockSpec((NB, DEGW), lambda i: (i, 0)),
    ],
    out_shape=[
        jax.ShapeDtypeStruct((N, F2), jnp.float32),
        jax.ShapeDtypeStruct((N, DEGW), jnp.float32),
    ],
)


def _tc_e_body(acc_ref, g1_ref, dinv_ref, w3_ref, b1_ref, g2_ref):
    dinv1 = dinv_ref[:, 0:1]
    out1 = jnp.maximum(
        dinv1 * (acc_ref[0] + acc_ref[1] + g1_ref[...]) + b1_ref[...], 0.0)
    g2_ref[...] = dinv1 * jnp.dot(
        out1, w3_ref[...], preferred_element_type=jnp.float32)


_tc_e = pl.pallas_call(
    _tc_e_body,
    grid=(NBLK,),
    in_specs=[
        pl.BlockSpec((NC, NB, F2), lambda i: (0, i, 0)),
        pl.BlockSpec((NB, F2), lambda i: (i, 0)),
        pl.BlockSpec((NB, DEGW), lambda i: (i, 0)),
        pl.BlockSpec((F2, F2), lambda i: (0, 0)),
        pl.BlockSpec((1, F2), lambda i: (0, 0)),
    ],
    out_specs=pl.BlockSpec((NB, F2), lambda i: (i, 0)),
    out_shape=jax.ShapeDtypeStruct((N, F2), jnp.float32),
)


def _tc_g_body(acc_ref, g2_ref, dinv_ref, batch_ref, b3_ref, wl1_ref, bl1_ref,
               wl2_ref, bl2_ref, out_ref, pool_ref, cnt_ref):
    i = pl.program_id(0)

    @pl.when(i == 0)
    def _init():
        pool_ref[...] = jnp.zeros_like(pool_ref)
        cnt_ref[...] = jnp.zeros_like(cnt_ref)

    dinv1 = dinv_ref[:, 0:1]
    h2 = dinv1 * (acc_ref[0] + acc_ref[1] + g2_ref[...]) + b3_ref[...]
    onehot = (batch_ref[...] == lax.broadcasted_iota(
        jnp.int32, (NB, G), 1)).astype(jnp.float32)
    dn = (((0,), (0,)), ((), ()))
    pool_ref[...] += lax.dot_general(
        onehot, h2, dn, preferred_element_type=jnp.float32)
    cnt_ref[...] += lax.dot_general(
        onehot, jnp.ones((NB, 1), jnp.float32), dn,
        preferred_element_type=jnp.float32)

    @pl.when(i == NBLK - 1)
    def _fin():
        pooled = pool_ref[...] / jnp.maximum(cnt_ref[...], 1.0)
        o1 = jnp.dot(pooled, wl1_ref[...],
                     preferred_element_type=jnp.float32) + bl1_ref[...]
        o2 = jnp.dot(o1, wl2_ref[...],
                     preferred_element_type=jnp.float32) + bl2_ref[...]
        m = jnp.max(o2, axis=-1, keepdims=True)
        ex = jnp.exp(o2 - m)
        out_ref[...] = ex / jnp.sum(ex, axis=-1, keepdims=True)


_tc_g = pl.pallas_call(
    _tc_g_body,
    grid=(NBLK,),
    in_specs=[
        pl.BlockSpec((NC, NB, F2), lambda i: (0, i, 0)),
        pl.BlockSpec((NB, F2), lambda i: (i, 0)),
        pl.BlockSpec((NB, DEGW), lambda i: (i, 0)),
        pl.BlockSpec((NB, 1), lambda i: (i, 0)),
        pl.BlockSpec((1, F2), lambda i: (0, 0)),
        pl.BlockSpec((F2, 32), lambda i: (0, 0)),
        pl.BlockSpec((1, 32), lambda i: (0, 0)),
        pl.BlockSpec((32, OUT), lambda i: (0, 0)),
        pl.BlockSpec((1, OUT), lambda i: (0, 0)),
    ],
    out_specs=pl.BlockSpec((G, OUT), lambda i: (0, 0)),
    out_shape=jax.ShapeDtypeStruct((G, OUT), jnp.float32),
    scratch_shapes=[
        pltpu.VMEM((G, F2), jnp.float32),
        pltpu.VMEM((G, 1), jnp.float32),
    ],
)


def kernel(x, edge_index, edge_attr, batch, W_rel, W_self, b_rgcn,
           W1, b1, W3, b3, Wl1, bl1, Wl2, bl2):
    src = edge_index[0]
    dst = edge_index[1]
    pad = E_PAD - E
    zpad = jnp.zeros((pad,), jnp.int32)
    gidx1 = jnp.concatenate([edge_attr * N + src, zpad]).reshape(NW, NCHUNK, CH)
    src_p = jnp.concatenate([src, zpad]).reshape(NW, NCHUNK, CH)
    didx = jnp.concatenate([dst, jnp.full((pad,), N, jnp.int32)]
                           ).reshape(NW, NCHUNK, CH)

    zacc1 = jnp.zeros((RPT, HID), jnp.float32)
    zacc2 = jnp.zeros((RPT, F2), jnp.float32)
    zdeg = jnp.zeros((RPT, DEGW), jnp.float32)
    ones_deg = jnp.ones((CH, DEGW), jnp.float32)

    hrel, selfp = _tc_a(x, W_rel, W_self, b_rgcn.reshape(1, HID))
    acc0, degp = _edge_pass_rgcn(hrel.reshape(R * N, HID), gidx1, didx,
                                 zacc1, zdeg, ones_deg)
    g1, dinv = _tc_c(acc0, selfp, degp, W1)
    acc1, = _edge_pass_gcn(g1, src_p, didx, zacc2)
    g2 = _tc_e(acc1, g1, dinv, W3, b1.reshape(1, F2))
    acc2, = _edge_pass_gcn(g2, src_p, didx, zacc2)
    return _tc_g(acc2, g2, dinv, batch.reshape(N, 1), b3.reshape(1, F2),
                 Wl1, bl1.reshape(1, 32), Wl2, bl2.reshape(1, OUT))


# speedup vs baseline: 2.1148x; 1.0203x over previous
import functools

import jax
import jax.numpy as jnp
from jax import lax
from jax.experimental import pallas as pl
from jax.experimental.pallas import tpu as pltpu
from jax.experimental.pallas import tpu_sc as plsc

N, E, D, R, G = 10000, 320000, 128, 4, 64
HID, F2, OUT = 32, 64, 10

NC, NS = 2, 16
NW = NC * NS
CH = 128
EPT = E // NW
NBUF = 8
NCHUNK = 80
WCH = 16
EPT_PAD = NCHUNK * CH
E_PAD = EPT_PAD * NW
NPAD = 10112
RPT = NPAD // NS
DEGW = 8
NB = 10000
NBLK = N // NB


def _edge_pass_body(with_deg, F, trows, staged, *args):
    if with_deg:
        (table_hbm, gidx_hbm, didx_hbm, zacc_hbm, zdeg_hbm, ones_hbm,
         acc_out, deg_out, gidx_v, didx_v, rows_v, rows_w, gsa, gsb,
         *maybe_tbl, acc_sh, ones_v, deg_sh) = args
    else:
        (table_hbm, gidx_hbm, didx_hbm, zacc_hbm,
         acc_out, gidx_v, didx_v, rows_v, rows_w, gsa, gsb,
         *maybe_tbl, acc_sh) = args
    c = lax.axis_index("c")
    s = lax.axis_index("s")
    wid = s * NC + c

    if not with_deg:
        pltpu.sync_copy(gidx_hbm.at[wid], gidx_v)
    pltpu.sync_copy(didx_hbm.at[wid], didx_v)
    if staged:
        table_sh = maybe_tbl[0]
        tslc = trows // NS
        pltpu.sync_copy(table_hbm.at[pl.ds(s * tslc, tslc)],
                        table_sh.at[pl.ds(s * tslc, tslc)])
    else:
        table_sh = table_hbm

    base = s * RPT
    pltpu.sync_copy(zacc_hbm, acc_sh.at[pl.ds(base, RPT)])
    if with_deg:
        pltpu.sync_copy(zdeg_hbm, deg_sh.at[pl.ds(base, RPT)])
        pltpu.sync_copy(ones_hbm, ones_v)

    plsc.subcore_barrier()

    def g_issue(j, buf, sem):
        pltpu.async_copy(table_sh.at[gidx_v.at[j]], buf, sem)

    def g_wait(buf, sem):
        pltpu.make_async_copy(table_sh.at[gidx_v.at[0]], buf, sem).wait()

    def scat(j, buf):
        pltpu.sync_copy(buf, acc_sh.at[didx_v.at[j]], add=True)

    if with_deg:
        def win(w, _):
            pltpu.sync_copy(gidx_hbm.at[wid, w], gidx_v)
            g_issue(0, rows_v, gsa)

            def pairw(p, _):
                l = p * 2
                j = w * WCH + l
                g_wait(rows_v, gsa)
                g_issue(jnp.minimum(l + 1, WCH - 1), rows_w, gsb)
                scat(j, rows_v)
                pltpu.sync_copy(ones_v, deg_sh.at[didx_v.at[j]], add=True)
                g_wait(rows_w, gsb)
                g_issue(jnp.minimum(l + 2, WCH - 1), rows_v, gsa)
                scat(j + 1, rows_w)
                pltpu.sync_copy(ones_v, deg_sh.at[didx_v.at[j + 1]],
                                add=True)
                return 0

            lax.fori_loop(0, WCH // 2, pairw, 0)
            g_wait(rows_v, gsa)
            return 0

        lax.fori_loop(0, NCHUNK // WCH, win, 0)
    else:
        g_issue(0, rows_v, gsa)

        def pair(p, _):
            j = p * 2
            g_wait(rows_v, gsa)
            g_issue(jnp.minimum(j + 1, NCHUNK - 1), rows_w, gsb)
            scat(j, rows_v)
            g_wait(rows_w, gsb)
            g_issue(jnp.minimum(j + 2, NCHUNK - 1), rows_v, gsa)
            scat(j + 1, rows_w)
            return 0

        lax.fori_loop(0, NCHUNK // 2, pair, 0)
        g_wait(rows_v, gsa)

    plsc.subcore_barrier()

    pltpu.sync_copy(acc_sh.at[pl.ds(base, RPT)], acc_out.at[c, pl.ds(base, RPT)])
    if with_deg:
        pltpu.sync_copy(deg_sh.at[pl.ds(base, RPT)], deg_out.at[c, pl.ds(base, RPT)])


def _make_edge_pass(F, trows, with_deg, staged):
    mesh = plsc.VectorSubcoreMesh(core_axis_name="c", subcore_axis_name="s")
    out_type = [jax.ShapeDtypeStruct((NC, NPAD, F), jnp.float32)]
    scratch = [
        pltpu.VMEM((WCH if with_deg else NCHUNK, CH), jnp.int32),
        pltpu.VMEM((NCHUNK, CH), jnp.int32),
        pltpu.VMEM((CH, F), jnp.float32),
        pltpu.VMEM((CH, F), jnp.float32),
        pltpu.SemaphoreType.DMA,
        pltpu.SemaphoreType.DMA,
    ]
    if staged:
        scratch.append(pltpu.VMEM_SHARED((trows, F), jnp.float32))
    scratch.append(pltpu.VMEM_SHARED((NPAD, F), jnp.float32))
    if with_deg:
        out_type.append(jax.ShapeDtypeStruct((NC, NPAD, DEGW), jnp.float32))
        scratch += [
            pltpu.VMEM((CH, DEGW), jnp.float32),
            pltpu.VMEM_SHARED((NPAD, DEGW), jnp.float32),
        ]
    return pl.kernel(
        functools.partial(_edge_pass_body, with_deg, F, trows, staged),
        out_type=out_type,
        mesh=mesh,
        scratch_types=scratch,
        compiler_params=pltpu.CompilerParams(use_tc_tiling_on_sc=False),
    )


_edge_pass_rgcn = _make_edge_pass(HID, R * N, True, True)
_edge_pass_gcn = _make_edge_pass(F2, N, False, True)


def _tc_a_body(x_ref, wrel_ref, wself_ref, brg_ref, hrel_ref, selfp_ref):
    xb = x_ref[...]
    for r in range(R):
        hrel_ref[r] = jnp.dot(xb, wrel_ref[r], preferred_element_type=jnp.float32)
    selfp_ref[...] = (
        jnp.dot(xb, wself_ref[...], preferred_element_type=jnp.float32)
        + brg_ref[...]
    )


_tc_a = pl.pallas_call(
    _tc_a_body,
    grid=(NBLK,),
    in_specs=[
        pl.BlockSpec((NB, D), lambda i: (i, 0)),
        pl.BlockSpec((R, D, HID), lambda i: (0, 0, 0)),
        pl.BlockSpec((D, HID), lambda i: (0, 0)),
        pl.BlockSpec((1, HID), lambda i: (0, 0)),
    ],
    out_specs=[
        pl.BlockSpec((R, NB, HID), lambda i: (0, i, 0)),
        pl.BlockSpec((NB, HID), lambda i: (i, 0)),
    ],
    out_shape=[
        jax.ShapeDtypeStruct((R, N, HID), jnp.float32),
        jax.ShapeDtypeStruct((N, HID), jnp.float32),
    ],
)


def _tc_c_body(acc_ref, selfp_ref, degp_ref, w1_ref, g1_ref, dinv_ref):
    h0 = jnp.maximum(acc_ref[0] + acc_ref[1] + selfp_ref[...], 0.0)
    deg = degp_ref[0] + degp_ref[1] + 1.0
    dinv = lax.rsqrt(deg)
    dinv_ref[...] = dinv
    g1_ref[...] = dinv[:, 0:1] * jnp.dot(
        h0, w1_ref[...], preferred_element_type=jnp.float32)


_tc_c = pl.pallas_call(
    _tc_c_body,
    grid=(NBLK,),
    in_specs=[
        pl.BlockSpec((NC, NB, HID), lambda i: (0, i, 0)),
        pl.BlockSpec((NB, HID), lambda i: (i, 0)),
        pl.BlockSpec((NC, NB, DEGW), lambda i: (0, i, 0)),
        pl.BlockSpec((HID, F2), lambda i: (0, 0)),
    ],
    out_specs=[
        pl.BlockSpec((NB, F2), lambda i: (i, 0)),
        pl.BlockSpec((NB, DEGW), lambda i: (i, 0)),
    ],
    out_shape=[
        jax.ShapeDtypeStruct((N, F2), jnp.float32),
        jax.ShapeDtypeStruct((N, DEGW), jnp.float32),
    ],
)


def _tc_e_body(acc_ref, g1_ref, dinv_ref, w3_ref, b1_ref, g2_ref):
    dinv1 = dinv_ref[:, 0:1]
    out1 = jnp.maximum(
        dinv1 * (acc_ref[0] + acc_ref[1] + g1_ref[...]) + b1_ref[...], 0.0)
    g2_ref[...] = dinv1 * jnp.dot(
        out1, w3_ref[...], preferred_element_type=jnp.float32)


_tc_e = pl.pallas_call(
    _tc_e_body,
    grid=(NBLK,),
    in_specs=[
        pl.BlockSpec((NC, NB, F2), lambda i: (0, i, 0)),
        pl.BlockSpec((NB, F2), lambda i: (i, 0)),
        pl.BlockSpec((NB, DEGW), lambda i: (i, 0)),
        pl.BlockSpec((F2, F2), lambda i: (0, 0)),
        pl.BlockSpec((1, F2), lambda i: (0, 0)),
    ],
    out_specs=pl.BlockSpec((NB, F2), lambda i: (i, 0)),
    out_shape=jax.ShapeDtypeStruct((N, F2), jnp.float32),
)


def _tc_g_body(acc_ref, g2_ref, dinv_ref, batch_ref, b3_ref, wl1_ref, bl1_ref,
               wl2_ref, bl2_ref, out_ref, pool_ref, cnt_ref):
    i = pl.program_id(0)

    @pl.when(i == 0)
    def _init():
        pool_ref[...] = jnp.zeros_like(pool_ref)
        cnt_ref[...] = jnp.zeros_like(cnt_ref)

    dinv1 = dinv_ref[:, 0:1]
    h2 = dinv1 * (acc_ref[0] + acc_ref[1] + g2_ref[...]) + b3_ref[...]
    onehot = (batch_ref[...] == lax.broadcasted_iota(
        jnp.int32, (NB, G), 1)).astype(jnp.float32)
    dn = (((0,), (0,)), ((), ()))
    pool_ref[...] += lax.dot_general(
        onehot, h2, dn, preferred_element_type=jnp.float32)
    cnt_ref[...] += lax.dot_general(
        onehot, jnp.ones((NB, 1), jnp.float32), dn,
        preferred_element_type=jnp.float32)

    @pl.when(i == NBLK - 1)
    def _fin():
        pooled = pool_ref[...] / jnp.maximum(cnt_ref[...], 1.0)
        o1 = jnp.dot(pooled, wl1_ref[...],
                     preferred_element_type=jnp.float32) + bl1_ref[...]
        o2 = jnp.dot(o1, wl2_ref[...],
                     preferred_element_type=jnp.float32) + bl2_ref[...]
        m = jnp.max(o2, axis=-1, keepdims=True)
        ex = jnp.exp(o2 - m)
        out_ref[...] = ex / jnp.sum(ex, axis=-1, keepdims=True)


_tc_g = pl.pallas_call(
    _tc_g_body,
    grid=(NBLK,),
    in_specs=[
        pl.BlockSpec((NC, NB, F2), lambda i: (0, i, 0)),
        pl.BlockSpec((NB, F2), lambda i: (i, 0)),
        pl.BlockSpec((NB, DEGW), lambda i: (i, 0)),
        pl.BlockSpec((NB, 1), lambda i: (i, 0)),
        pl.BlockSpec((1, F2), lambda i: (0, 0)),
        pl.BlockSpec((F2, 32), lambda i: (0, 0)),
        pl.BlockSpec((1, 32), lambda i: (0, 0)),
        pl.BlockSpec((32, OUT), lambda i: (0, 0)),
        pl.BlockSpec((1, OUT), lambda i: (0, 0)),
    ],
    out_specs=pl.BlockSpec((G, OUT), lambda i: (0, 0)),
    out_shape=jax.ShapeDtypeStruct((G, OUT), jnp.float32),
    scratch_shapes=[
        pltpu.VMEM((G, F2), jnp.float32),
        pltpu.VMEM((G, 1), jnp.float32),
    ],
)


def kernel(x, edge_index, edge_attr, batch, W_rel, W_self, b_rgcn,
           W1, b1, W3, b3, Wl1, bl1, Wl2, bl2):
    src = edge_index[0]
    dst = edge_index[1]
    pad = E_PAD - E
    zpad = jnp.zeros((pad,), jnp.int32)
    gidx1 = jnp.concatenate([edge_attr * N + src, zpad]).reshape(
        NW, NCHUNK // WCH, WCH, CH)
    src_p = jnp.concatenate([src, zpad]).reshape(NW, NCHUNK, CH)
    didx = jnp.concatenate([dst, jnp.full((pad,), N, jnp.int32)]
                           ).reshape(NW, NCHUNK, CH)

    zacc1 = jnp.zeros((RPT, HID), jnp.float32)
    zacc2 = jnp.zeros((RPT, F2), jnp.float32)
    zdeg = jnp.zeros((RPT, DEGW), jnp.float32)
    ones_deg = jnp.ones((CH, DEGW), jnp.float32)

    hrel, selfp = _tc_a(x, W_rel, W_self, b_rgcn.reshape(1, HID))
    acc0, degp = _edge_pass_rgcn(hrel.reshape(R * N, HID), gidx1, didx,
                                 zacc1, zdeg, ones_deg)
    g1, dinv = _tc_c(acc0, selfp, degp, W1)
    acc1, = _edge_pass_gcn(g1, src_p, didx, zacc2)
    g2 = _tc_e(acc1, g1, dinv, W3, b1.reshape(1, F2))
    acc2, = _edge_pass_gcn(g2, src_p, didx, zacc2)
    return _tc_g(acc2, g2, dinv, batch.reshape(N, 1), b3.reshape(1, F2),
                 Wl1, bl1.reshape(1, 32), Wl2, bl2.reshape(1, OUT))
